# baseline (speedup 1.0000x reference)
import jax
import jax.numpy as jnp
from jax import lax
from jax.experimental import pallas as pl
from jax.experimental.pallas import tpu as pltpu
from jax.experimental.pallas import tpu_sc as plsc

_NC = 2
_NS = 16
_NW = _NC * _NS
_NBUF = 2


def _emb_body(idx_hbm, table_hbm, out_hbm,
              idx0, idx1, rows0, rows1,
              isem0, isem1, gsem0, gsem1, wsem0, wsem1):
    idx_v = (idx0, idx1)
    rows_v = (rows0, rows1)
    isem = (isem0, isem1)
    gsem = (gsem0, gsem1)
    wsem = (wsem0, wsem1)

    batch = idx_hbm.shape[0]
    rows_per_w = batch // _NW
    wid = lax.axis_index("s") * _NC + lax.axis_index("c")
    base_w = wid * rows_per_w

    for p in range(_NBUF):
        pltpu.async_copy(idx_hbm.at[base_w + p], idx_v[p], isem[p])
    for p in range(_NBUF):
        pltpu.make_async_copy(idx_hbm.at[base_w + p], idx_v[p], isem[p]).wait()
        pltpu.async_copy(table_hbm.at[idx_v[p]], rows_v[p], gsem[p])

    def body(g, carry):
        for p in range(_NBUF):
            bold = base_w + (g - 1) * _NBUF + p
            bnew = base_w + g * _NBUF + p
            pltpu.make_async_copy(table_hbm.at[idx_v[p]], rows_v[p],
                                  gsem[p]).wait()
            pltpu.async_copy(idx_hbm.at[bnew], idx_v[p], isem[p])
            pltpu.async_copy(rows_v[p], out_hbm.at[bold], wsem[p])
            pltpu.make_async_copy(rows_v[p], out_hbm.at[bold], wsem[p]).wait()
            pltpu.make_async_copy(idx_hbm.at[bnew], idx_v[p], isem[p]).wait()
            pltpu.async_copy(table_hbm.at[idx_v[p]], rows_v[p], gsem[p])
        return carry

    lax.fori_loop(1, rows_per_w // _NBUF, body, 0)

    for p in range(_NBUF):
        bold = base_w + rows_per_w - _NBUF + p
        pltpu.make_async_copy(table_hbm.at[idx_v[p]], rows_v[p],
                              gsem[p]).wait()
        pltpu.async_copy(rows_v[p], out_hbm.at[bold], wsem[p])
    for p in range(_NBUF):
        bold = base_w + rows_per_w - _NBUF + p
        pltpu.make_async_copy(rows_v[p], out_hbm.at[bold], wsem[p]).wait()


def kernel(input, table):
    batch, hist = input.shape
    dim = table.shape[1]
    mesh = plsc.VectorSubcoreMesh(core_axis_name="c", subcore_axis_name="s")
    f = pl.kernel(
        _emb_body,
        out_type=jax.ShapeDtypeStruct((batch, hist, dim), jnp.float32),
        mesh=mesh,
        scratch_types=[
            pltpu.VMEM((hist,), jnp.int32),
            pltpu.VMEM((hist,), jnp.int32),
            pltpu.VMEM((hist, dim), jnp.float32),
            pltpu.VMEM((hist, dim), jnp.float32),
            pltpu.SemaphoreType.DMA,
            pltpu.SemaphoreType.DMA,
            pltpu.SemaphoreType.DMA,
            pltpu.SemaphoreType.DMA,
            pltpu.SemaphoreType.DMA,
            pltpu.SemaphoreType.DMA,
        ],
        compiler_params=pltpu.CompilerParams(use_tc_tiling_on_sc=False),
    )
    return f(input, table)
